# baseline (speedup 1.0000x reference)
import functools

import jax
import jax.numpy as jnp
from jax.experimental import pallas as pl
from jax.experimental.pallas import tpu as pltpu


_BS = 8
_N = 2048
_NC = 32
_HR = _N // 2


def _mse_kernel(x1_ref, x2_ref, starts_ref, ends_ref, out_ref, acc_ref):
    b = pl.program_id(0)

    @pl.when(b == 0)
    def _init():
        acc_ref[...] = jnp.zeros_like(acc_ref)

    starts = starts_ref[0, 0, :].reshape(1, _NC)
    ends = ends_ref[0, 0, :].reshape(1, _NC)

    ones = jnp.ones((8, _HR), jnp.float32)

    def stream(x_ref, base):
        rows = jax.lax.broadcasted_iota(jnp.int32, (_HR, _NC), 0) + base
        inb = (rows >= starts) & (rows < ends)
        lo = jnp.min(jnp.where(inb, starts, _N), axis=1, keepdims=True)
        hi = jnp.max(jnp.where(inb, ends, 0), axis=1, keepdims=True)
        cols = jax.lax.broadcasted_iota(jnp.int32, (_HR, _N), 1)
        rel = jax.lax.bitcast_convert_type(cols - lo, jnp.uint32)
        width = jax.lax.bitcast_convert_type(hi - lo, jnp.uint32)
        pred = rel < width
        x = x_ref[0]
        diff = jnp.where(pred, x - 1.0, x)
        d2 = diff * diff
        return jax.lax.dot_general(
            ones, d2, (((1,), (0,)), ((), ())), preferred_element_type=jnp.float32
        )

    acc_ref[...] += stream(x1_ref, 0) + stream(x2_ref, _HR)

    @pl.when(b == _BS - 1)
    def _fin():
        out_ref[...] = jnp.sum(acc_ref[...]).reshape(1, 1)


@functools.partial(jax.jit, static_argnames=())
def _loss(raw_scores, starts, ends):
    total = pl.pallas_call(
        _mse_kernel,
        grid=(_BS,),
        in_specs=[
            pl.BlockSpec((1, _HR, _N), lambda b: (b, 0, 0)),
            pl.BlockSpec((1, _HR, _N), lambda b: (b, 1, 0)),
            pl.BlockSpec((1, 1, _NC), lambda b: (b, 0, 0)),
            pl.BlockSpec((1, 1, _NC), lambda b: (b, 0, 0)),
        ],
        out_specs=pl.BlockSpec((1, 1), lambda b: (0, 0)),
        out_shape=jax.ShapeDtypeStruct((1, 1), jnp.float32),
        scratch_shapes=[pltpu.VMEM((8, _N), jnp.float32)],
    )(raw_scores, raw_scores, starts, ends)
    return total[0, 0] / jnp.float32(_BS * _N * _N * 8)


def kernel(raw_scores, cluster_sizes):
    cs = cluster_sizes.astype(jnp.int32)
    starts = jnp.concatenate(
        [jnp.zeros((_BS, 1), dtype=jnp.int32), cs[:, :-1]], axis=1
    ).reshape(_BS, 1, _NC)
    ends = starts + cs.reshape(_BS, 1, _NC)
    return _loss(raw_scores, starts, ends)
